# baseline (speedup 1.0000x reference)
import functools

import jax
import jax.numpy as jnp
from jax import lax
from jax.experimental import pallas as pl
from jax.experimental.pallas import tpu as pltpu
from jax.experimental.pallas import tpu_sc as plsc

N_NODES = 10000
N_EDGES = 160000
D_FEAT = 256
N_CLASSES = 64
K = 10
ALPHA = 0.1

NS = 16
NN = 10240
NPT = NN // NS
CH = 128
EPT = N_EDGES // NS
NCH = 80
EPAD = NCH * CH
DUMMY = N_NODES
D = N_CLASSES
LPR = D // 16
NCC = NPT // CH
OMA = 1.0 - ALPHA


def _matmul_body(x_ref, w_ref, h_ref, ah_ref):
    h = jnp.dot(x_ref[...], w_ref[...], preferred_element_type=jnp.float32)
    h_ref[...] = h
    ah_ref[...] = h * ALPHA


def _project(feat_pad, wt):
    blk = 1024
    return pl.pallas_call(
        _matmul_body,
        grid=(NN // blk,),
        in_specs=[
            pl.BlockSpec((blk, D_FEAT), lambda i: (i, 0)),
            pl.BlockSpec((D_FEAT, D), lambda i: (0, 0)),
        ],
        out_specs=[
            pl.BlockSpec((blk, D), lambda i: (i, 0)),
            pl.BlockSpec((blk, D), lambda i: (i, 0)),
        ],
        out_shape=[
            jax.ShapeDtypeStruct((NN, D), jnp.float32),
            jax.ShapeDtypeStruct((NN, D), jnp.float32),
        ],
    )(feat_pad, wt)


def _prep_body(cs_ref, cd_ref, ah_ref, hp_ref, normc_ref, ahn_ref, xs0_ref):
    ns = lax.rsqrt(jnp.maximum(cs_ref[:, :1], 1.0))
    nv = OMA * lax.rsqrt(jnp.maximum(cd_ref[:, :1], 1.0))
    normc_ref[...] = jnp.concatenate(
        [jnp.broadcast_to(ns * nv, (NN, 16)),
         jnp.broadcast_to(nv, (NN, 16))], axis=1)
    ahn_ref[...] = ah_ref[...] / nv
    xs0_ref[...] = ns * hp_ref[...]


def _prep(cs, cd, ah, hp):
    return pl.pallas_call(
        _prep_body,
        out_shape=[
            jax.ShapeDtypeStruct((NN, 32), jnp.float32),
            jax.ShapeDtypeStruct((NN, D), jnp.float32),
            jax.ShapeDtypeStruct((NN, D), jnp.float32),
        ],
    )(cs, cd, ah, hp)


def _sc_degrees_body(src_hbm, dst_hbm, cs_hbm, cd_hbm,
                     src_v, dst_v, ones16, zbuf16, cs_sh, cd_sh):
    cid = lax.axis_index("c")
    tid = lax.axis_index("s")

    @pl.when(cid == 0)
    def _body():
        row0 = tid * NPT
        zvec = jnp.zeros((16,), jnp.float32)
        ovec = jnp.ones((16,), jnp.float32)

        pltpu.sync_copy(src_hbm.at[tid], src_v)
        pltpu.sync_copy(dst_hbm.at[tid], dst_v)

        def _fill_o(i, _):
            ones16[i, pl.ds(0, 16)] = ovec
            return _
        lax.fori_loop(0, CH, _fill_o, None)

        def _fill_z16(i, _):
            zbuf16[i, pl.ds(0, 16)] = zvec
            return _
        lax.fori_loop(0, NPT, _fill_z16, None)

        pltpu.sync_copy(zbuf16, cs_sh.at[pl.ds(row0, NPT)])
        pltpu.sync_copy(zbuf16, cd_sh.at[pl.ds(row0, NPT)])
        plsc.subcore_barrier()

        def _deg_chunk(j, _):
            pltpu.sync_copy(ones16, cs_sh.at[src_v.at[j]], add=True)
            pltpu.sync_copy(ones16, cd_sh.at[dst_v.at[j]], add=True)
            return _
        lax.fori_loop(0, NCH, _deg_chunk, None)
        plsc.subcore_barrier()

        pltpu.sync_copy(cs_sh.at[pl.ds(row0, NPT)],
                        cs_hbm.at[pl.ds(row0, NPT)])
        pltpu.sync_copy(cd_sh.at[pl.ds(row0, NPT)],
                        cd_hbm.at[pl.ds(row0, NPT)])


def _sc_degrees(srcp, dstp):
    mesh = plsc.VectorSubcoreMesh(core_axis_name="c", subcore_axis_name="s")
    fn = functools.partial(
        pl.kernel,
        mesh=mesh,
        compiler_params=pltpu.CompilerParams(use_tc_tiling_on_sc=False),
        out_type=[
            jax.ShapeDtypeStruct((NN, 16), jnp.float32),
            jax.ShapeDtypeStruct((NN, 16), jnp.float32),
        ],
        scratch_types=[
            pltpu.VMEM((NCH, CH), jnp.int32),
            pltpu.VMEM((NCH, CH), jnp.int32),
            pltpu.VMEM((CH, 16), jnp.float32),
            pltpu.VMEM((NPT, 16), jnp.float32),
            pltpu.VMEM_SHARED((NN, 16), jnp.float32),
            pltpu.VMEM_SHARED((NN, 16), jnp.float32),
        ],
    )(_sc_degrees_body)
    return fn(srcp, dstp)


def _sc_propagate_body(ahn_hbm, normc_hbm, src_hbm, dst_hbm, xs0_hbm,
                       out_hbm, xs_hbm,
                       src_v, dst_v, rowbuf, aggc, outc, nstri,
                       agg_sh, sem, isem):
    cid = lax.axis_index("c")
    tid = lax.axis_index("s")

    @pl.when(cid == 0)
    def _body():
        row0 = tid * NPT

        pltpu.sync_copy(src_hbm.at[tid], src_v)
        pltpu.sync_copy(dst_hbm.at[tid], dst_v)
        pltpu.sync_copy(normc_hbm.at[pl.ds(row0, NPT)], nstri)
        pltpu.sync_copy(ahn_hbm.at[pl.ds(row0, NPT)],
                        agg_sh.at[pl.ds(row0, NPT)])
        plsc.subcore_barrier()

        def _scatter_phase(src_ref):
            def _chunk(j, _):
                pltpu.async_copy(src_ref.at[src_v.at[j]], rowbuf, sem).wait()
                pltpu.sync_copy(rowbuf, agg_sh.at[dst_v.at[j]], add=True)
                return _
            lax.fori_loop(0, NCH, _chunk, None)
            plsc.subcore_barrier()

        def _update_phase(last):
            def _upd_chunk(c, _):
                r0 = row0 + c * CH
                pltpu.sync_copy(agg_sh.at[pl.ds(r0, CH)], aggc)
                pltpu.async_copy(ahn_hbm.at[pl.ds(r0, CH)],
                                 agg_sh.at[pl.ds(r0, CH)], isem)

                def _rows(r, _):
                    i = c * CH + r
                    snv = nstri[i, pl.ds(0, 16)]
                    nv = nstri[i, pl.ds(16, 16)]
                    f = jnp.where(last, nv, snv)
                    for v in range(LPR):
                        sl = pl.ds(v * 16, 16)
                        outc[r, sl] = f * aggc[r, sl]
                    return _
                lax.fori_loop(0, CH, _rows, None)

                @pl.when(jnp.logical_not(last))
                def _():
                    pltpu.sync_copy(outc, xs_hbm.at[pl.ds(r0, CH)])

                @pl.when(last)
                def _():
                    pltpu.sync_copy(outc, out_hbm.at[pl.ds(r0, CH)])
                return _
            lax.fori_loop(0, NCC, _upd_chunk, None)

            def _drain(c, _):
                r0 = row0 + c * CH
                pltpu.make_async_copy(ahn_hbm.at[pl.ds(r0, CH)],
                                      agg_sh.at[pl.ds(r0, CH)], isem).wait()
                return _
            lax.fori_loop(0, NCC, _drain, None)
            plsc.subcore_barrier()

        _scatter_phase(xs0_hbm)
        _update_phase(jnp.bool_(False))

        def _iter(k, _):
            _scatter_phase(xs_hbm)
            _update_phase(k == K - 2)
            return _
        lax.fori_loop(0, K - 1, _iter, None)


def _sc_propagate(ahn, normc, srcp, dstp, xs0):
    mesh = plsc.VectorSubcoreMesh(core_axis_name="c", subcore_axis_name="s")
    fn = functools.partial(
        pl.kernel,
        mesh=mesh,
        compiler_params=pltpu.CompilerParams(use_tc_tiling_on_sc=False),
        out_type=[
            jax.ShapeDtypeStruct((NN, D), jnp.float32),
            jax.ShapeDtypeStruct((NN, D), jnp.float32),
        ],
        scratch_types=[
            pltpu.VMEM((NCH, CH), jnp.int32),
            pltpu.VMEM((NCH, CH), jnp.int32),
            pltpu.VMEM((CH, D), jnp.float32),
            pltpu.VMEM((CH, D), jnp.float32),
            pltpu.VMEM((CH, D), jnp.float32),
            pltpu.VMEM((NPT, 32), jnp.float32),
            pltpu.VMEM_SHARED((NN, D), jnp.float32),
            pltpu.SemaphoreType.DMA,
            pltpu.SemaphoreType.DMA,
        ],
    )(_sc_propagate_body)
    return fn(ahn, normc, srcp, dstp, xs0)


def kernel(features, edge_index, W):
    src = edge_index[0].astype(jnp.int32).reshape(NS, EPT)
    dst = edge_index[1].astype(jnp.int32).reshape(NS, EPT)
    pad = ((0, 0), (0, EPAD - EPT))
    srcp = jnp.pad(src, pad, constant_values=DUMMY).reshape(NS, NCH, CH)
    dstp = jnp.pad(dst, pad, constant_values=DUMMY).reshape(NS, NCH, CH)

    feat_pad = jnp.pad(features, ((0, NN - N_NODES), (0, 0)))
    hp, ah = _project(feat_pad, W.T)

    cs, cd = _sc_degrees(srcp, dstp)
    normc, ahn, xs0 = _prep(cs, cd, ah, hp)

    out_pad, _ = _sc_propagate(ahn, normc, srcp, dstp, xs0)
    return out_pad[:N_NODES]


# speedup vs baseline: 1.3513x; 1.3513x over previous
import functools

import jax
import jax.numpy as jnp
from jax import lax
from jax.experimental import pallas as pl
from jax.experimental.pallas import tpu as pltpu
from jax.experimental.pallas import tpu_sc as plsc

N_NODES = 10000
N_EDGES = 160000
D_FEAT = 256
N_CLASSES = 64
K = 10
ALPHA = 0.1

NS = 16
NN = 10240
NPT = NN // NS
CH = 128
EPT = N_EDGES // NS
NCH = 80
EPAD = NCH * CH
DUMMY = N_NODES
D = N_CLASSES
DC = D // 2
LPR = DC // 16
NCC = NPT // CH
OMA = 1.0 - ALPHA


def _matmul_body(x_ref, w_ref, h_ref, ah_ref):
    h = jnp.dot(x_ref[...], w_ref[...], preferred_element_type=jnp.float32)
    h_ref[...] = h
    ah_ref[...] = h * ALPHA


def _project(feat_pad, wt):
    blk = 1024
    return pl.pallas_call(
        _matmul_body,
        grid=(NN // blk,),
        in_specs=[
            pl.BlockSpec((blk, D_FEAT), lambda i: (i, 0)),
            pl.BlockSpec((D_FEAT, D), lambda i: (0, 0)),
        ],
        out_specs=[
            pl.BlockSpec((blk, D), lambda i: (i, 0)),
            pl.BlockSpec((blk, D), lambda i: (i, 0)),
        ],
        out_shape=[
            jax.ShapeDtypeStruct((NN, D), jnp.float32),
            jax.ShapeDtypeStruct((NN, D), jnp.float32),
        ],
    )(feat_pad, wt)


def _prep_body(cnt_ref, ah_ref, hp_ref, normc_ref, ahh_ref, xs0_ref):
    ns = lax.rsqrt(jnp.maximum(cnt_ref[0, :, :1], 1.0))
    nv = OMA * lax.rsqrt(jnp.maximum(cnt_ref[1, :, :1], 1.0))
    normc_ref[...] = jnp.concatenate(
        [jnp.broadcast_to(ns, (NN, 16)),
         jnp.broadcast_to(nv, (NN, 16))], axis=1)
    ah = ah_ref[...]
    ahh_ref[0] = ah[:, :DC]
    ahh_ref[1] = ah[:, DC:]
    xs0 = ns * hp_ref[...]
    xs0_ref[0] = xs0[:, :DC]
    xs0_ref[1] = xs0[:, DC:]


def _prep(cnt, ah, hp):
    return pl.pallas_call(
        _prep_body,
        out_shape=[
            jax.ShapeDtypeStruct((NN, 32), jnp.float32),
            jax.ShapeDtypeStruct((2, NN, DC), jnp.float32),
            jax.ShapeDtypeStruct((2, NN, DC), jnp.float32),
        ],
    )(cnt, ah, hp)


def _sc_degrees_body(sd_hbm, cnt_hbm, idx_v, ones16, zbuf16, cnt_sh):
    cid = lax.axis_index("c")
    tid = lax.axis_index("s")
    row0 = tid * NPT
    zvec = jnp.zeros((16,), jnp.float32)
    ovec = jnp.ones((16,), jnp.float32)

    pltpu.sync_copy(sd_hbm.at[cid, tid], idx_v)

    def _fill_o(i, _):
        ones16[i, pl.ds(0, 16)] = ovec
        return _
    lax.fori_loop(0, CH, _fill_o, None)

    def _fill_z16(i, _):
        zbuf16[i, pl.ds(0, 16)] = zvec
        return _
    lax.fori_loop(0, NPT, _fill_z16, None)

    pltpu.sync_copy(zbuf16, cnt_sh.at[pl.ds(row0, NPT)])
    plsc.subcore_barrier()

    def _deg_chunk(j, _):
        pltpu.sync_copy(ones16, cnt_sh.at[idx_v.at[j]], add=True)
        return _
    lax.fori_loop(0, NCH, _deg_chunk, None)
    plsc.subcore_barrier()

    pltpu.sync_copy(cnt_sh.at[pl.ds(row0, NPT)],
                    cnt_hbm.at[cid, pl.ds(row0, NPT)])


def _sc_degrees(srcdst):
    mesh = plsc.VectorSubcoreMesh(core_axis_name="c", subcore_axis_name="s")
    fn = functools.partial(
        pl.kernel,
        mesh=mesh,
        compiler_params=pltpu.CompilerParams(use_tc_tiling_on_sc=False),
        out_type=jax.ShapeDtypeStruct((2, NN, 16), jnp.float32),
        scratch_types=[
            pltpu.VMEM((NCH, CH), jnp.int32),
            pltpu.VMEM((CH, 16), jnp.float32),
            pltpu.VMEM((NPT, 16), jnp.float32),
            pltpu.VMEM_SHARED((NN, 16), jnp.float32),
        ],
    )(_sc_degrees_body)
    return fn(srcdst)


def _sc_propagate_body(ahh_hbm, normc_hbm, src_hbm, dst_hbm, xs0_hbm,
                       out_hbm, xs_hbm,
                       src_v, dst_v, rowbuf, aggc, ahc, outc, zeroc,
                       nstri, agg_sh, sem):
    cid = lax.axis_index("c")
    tid = lax.axis_index("s")
    row0 = tid * NPT
    zvec = jnp.zeros((16,), jnp.float32)

    pltpu.sync_copy(src_hbm.at[tid], src_v)
    pltpu.sync_copy(dst_hbm.at[tid], dst_v)
    pltpu.sync_copy(normc_hbm.at[pl.ds(row0, NPT)], nstri)
    pltpu.sync_copy(ahh_hbm.at[cid, pl.ds(row0, NPT)], ahc)

    def _fill_zc(i, _):
        zeroc[i // LPR, pl.ds((i % LPR) * 16, 16)] = zvec
        return _
    lax.fori_loop(0, CH * LPR, _fill_zc, None)

    def _zero_agg(c, _):
        pltpu.sync_copy(zeroc, agg_sh.at[pl.ds(row0 + c * CH, CH)])
        return _
    lax.fori_loop(0, NCC, _zero_agg, None)
    plsc.subcore_barrier()

    def _scatter_phase(src_ref):
        def _chunk(j, _):
            pltpu.async_copy(src_ref.at[cid].at[src_v.at[j]],
                             rowbuf, sem).wait()
            pltpu.sync_copy(rowbuf, agg_sh.at[dst_v.at[j]], add=True)
            return _
        lax.fori_loop(0, NCH, _chunk, None)
        plsc.subcore_barrier()

    def _update_phase(last):
        def _upd_chunk(c, _):
            r0 = row0 + c * CH
            pltpu.sync_copy(agg_sh.at[pl.ds(r0, CH)], aggc)
            pltpu.sync_copy(zeroc, agg_sh.at[pl.ds(r0, CH)])

            def _rows(r, _):
                i = c * CH + r
                ns = nstri[i, pl.ds(0, 16)]
                nv = nstri[i, pl.ds(16, 16)]
                f = jnp.where(last, 1.0, ns)
                for v in range(LPR):
                    sl = pl.ds(v * 16, 16)
                    outc[r, sl] = f * (ahc[i, sl] + nv * aggc[r, sl])
                return _
            lax.fori_loop(0, CH, _rows, None)

            @pl.when(jnp.logical_not(last))
            def _():
                pltpu.sync_copy(outc, xs_hbm.at[cid, pl.ds(r0, CH)])

            @pl.when(last)
            def _():
                pltpu.sync_copy(outc, out_hbm.at[cid, pl.ds(r0, CH)])
            return _
        lax.fori_loop(0, NCC, _upd_chunk, None)
        plsc.subcore_barrier()

    _scatter_phase(xs0_hbm)
    _update_phase(jnp.bool_(False))

    def _iter(k, _):
        _scatter_phase(xs_hbm)
        _update_phase(k == K - 2)
        return _
    lax.fori_loop(0, K - 1, _iter, None)


def _sc_propagate(ahh, normc, srcp, dstp, xs0):
    mesh = plsc.VectorSubcoreMesh(core_axis_name="c", subcore_axis_name="s")
    fn = functools.partial(
        pl.kernel,
        mesh=mesh,
        compiler_params=pltpu.CompilerParams(use_tc_tiling_on_sc=False),
        out_type=[
            jax.ShapeDtypeStruct((2, NN, DC), jnp.float32),
            jax.ShapeDtypeStruct((2, NN, DC), jnp.float32),
        ],
        scratch_types=[
            pltpu.VMEM((NCH, CH), jnp.int32),
            pltpu.VMEM((NCH, CH), jnp.int32),
            pltpu.VMEM((CH, DC), jnp.float32),
            pltpu.VMEM((CH, DC), jnp.float32),
            pltpu.VMEM((NPT, DC), jnp.float32),
            pltpu.VMEM((CH, DC), jnp.float32),
            pltpu.VMEM((CH, DC), jnp.float32),
            pltpu.VMEM((NPT, 32), jnp.float32),
            pltpu.VMEM_SHARED((NN, DC), jnp.float32),
            pltpu.SemaphoreType.DMA,
        ],
    )(_sc_propagate_body)
    return fn(ahh, normc, srcp, dstp, xs0)


def kernel(features, edge_index, W):
    src = edge_index[0].astype(jnp.int32).reshape(NS, EPT)
    dst = edge_index[1].astype(jnp.int32).reshape(NS, EPT)
    pad = ((0, 0), (0, EPAD - EPT))
    srcp = jnp.pad(src, pad, constant_values=DUMMY).reshape(NS, NCH, CH)
    dstp = jnp.pad(dst, pad, constant_values=DUMMY).reshape(NS, NCH, CH)
    srcdst = jnp.stack([srcp, dstp])

    feat_pad = jnp.pad(features, ((0, NN - N_NODES), (0, 0)))
    hp, ah = _project(feat_pad, W.T)

    cnt = _sc_degrees(srcdst)
    normc, ahh, xs0 = _prep(cnt, ah, hp)

    out_pad, _ = _sc_propagate(ahh, normc, srcp, dstp, xs0)
    return jnp.concatenate([out_pad[0], out_pad[1]], axis=1)[:N_NODES]


# speedup vs baseline: 1.7040x; 1.2611x over previous
import functools

import jax
import jax.numpy as jnp
from jax import lax
from jax.experimental import pallas as pl
from jax.experimental.pallas import tpu as pltpu
from jax.experimental.pallas import tpu_sc as plsc

N_NODES = 10000
N_EDGES = 160000
D_FEAT = 256
N_CLASSES = 64
K = 10
ALPHA = 0.1

NS = 16
NN = 10240
NPT = NN // NS
CH = 128
EPT = N_EDGES // NS
NCH = 80
EPAD = NCH * CH
DUMMY = N_NODES
D = N_CLASSES
DC = D // 2
LPR = DC // 16
NCC = NPT // CH
OMA = 1.0 - ALPHA


def _matmul_body(x_ref, w_ref, h_ref, ah_ref):
    h = jnp.dot(x_ref[...], w_ref[...], preferred_element_type=jnp.float32)
    h_ref[...] = h
    ah_ref[...] = h * ALPHA


def _project(feat_pad, wt):
    blk = 1024
    return pl.pallas_call(
        _matmul_body,
        grid=(NN // blk,),
        in_specs=[
            pl.BlockSpec((blk, D_FEAT), lambda i: (i, 0)),
            pl.BlockSpec((D_FEAT, D), lambda i: (0, 0)),
        ],
        out_specs=[
            pl.BlockSpec((blk, D), lambda i: (i, 0)),
            pl.BlockSpec((blk, D), lambda i: (i, 0)),
        ],
        out_shape=[
            jax.ShapeDtypeStruct((NN, D), jnp.float32),
            jax.ShapeDtypeStruct((NN, D), jnp.float32),
        ],
    )(feat_pad, wt)


def _prep_body(cnt_ref, ah_ref, hp_ref, normc_ref, ahh_ref, xs0_ref):
    ns = lax.rsqrt(jnp.maximum(cnt_ref[0, :, :1], 1.0))
    nv = OMA * lax.rsqrt(jnp.maximum(cnt_ref[1, :, :1], 1.0))
    normc_ref[...] = jnp.concatenate(
        [jnp.broadcast_to(ns, (NN, 16)),
         jnp.broadcast_to(nv, (NN, 16))], axis=1)
    ah = ah_ref[...]
    ahh_ref[0] = ah[:, :DC]
    ahh_ref[1] = ah[:, DC:]
    xs0 = ns * hp_ref[...]
    xs0_ref[0] = xs0[:, :DC]
    xs0_ref[1] = xs0[:, DC:]


def _prep(cnt, ah, hp):
    return pl.pallas_call(
        _prep_body,
        out_shape=[
            jax.ShapeDtypeStruct((NN, 32), jnp.float32),
            jax.ShapeDtypeStruct((2, NN, DC), jnp.float32),
            jax.ShapeDtypeStruct((2, NN, DC), jnp.float32),
        ],
    )(cnt, ah, hp)


def _sc_degrees_body(sd_hbm, cnt_hbm, idx_v, ones16, zbuf16, cnt_sh):
    cid = lax.axis_index("c")
    tid = lax.axis_index("s")
    row0 = tid * NPT
    zvec = jnp.zeros((16,), jnp.float32)
    ovec = jnp.ones((16,), jnp.float32)

    pltpu.sync_copy(sd_hbm.at[cid, tid], idx_v)

    def _fill_o(i, _):
        ones16[i, pl.ds(0, 16)] = ovec
        return _
    lax.fori_loop(0, CH, _fill_o, None)

    def _fill_z16(i, _):
        zbuf16[i, pl.ds(0, 16)] = zvec
        return _
    lax.fori_loop(0, NPT, _fill_z16, None)

    pltpu.sync_copy(zbuf16, cnt_sh.at[pl.ds(row0, NPT)])
    plsc.subcore_barrier()

    def _deg_chunk(j, _):
        pltpu.sync_copy(ones16, cnt_sh.at[idx_v.at[j]], add=True)
        return _
    lax.fori_loop(0, NCH, _deg_chunk, None)
    plsc.subcore_barrier()

    pltpu.sync_copy(cnt_sh.at[pl.ds(row0, NPT)],
                    cnt_hbm.at[cid, pl.ds(row0, NPT)])


def _sc_degrees(srcdst):
    mesh = plsc.VectorSubcoreMesh(core_axis_name="c", subcore_axis_name="s")
    fn = functools.partial(
        pl.kernel,
        mesh=mesh,
        compiler_params=pltpu.CompilerParams(use_tc_tiling_on_sc=False),
        out_type=jax.ShapeDtypeStruct((2, NN, 16), jnp.float32),
        scratch_types=[
            pltpu.VMEM((NCH, CH), jnp.int32),
            pltpu.VMEM((CH, 16), jnp.float32),
            pltpu.VMEM((NPT, 16), jnp.float32),
            pltpu.VMEM_SHARED((NN, 16), jnp.float32),
        ],
    )(_sc_degrees_body)
    return fn(srcdst)


def _sc_propagate_body(ahh_hbm, normc_hbm, src_hbm, dst_hbm, xs0_hbm,
                       out_hbm, xs_hbm,
                       src_v, dst_v, rowbuf, rowbuf1, aggc, ahc, outc, zeroc,
                       nstri, agg_sh, sem, sem1, ssem, ssem1):
    cid = lax.axis_index("c")
    tid = lax.axis_index("s")
    row0 = tid * NPT
    zvec = jnp.zeros((16,), jnp.float32)

    pltpu.sync_copy(src_hbm.at[tid], src_v)
    pltpu.sync_copy(dst_hbm.at[tid], dst_v)
    pltpu.sync_copy(normc_hbm.at[pl.ds(row0, NPT)], nstri)
    pltpu.sync_copy(ahh_hbm.at[cid, pl.ds(row0, NPT)], ahc)

    def _fill_zc(i, _):
        zeroc[i // LPR, pl.ds((i % LPR) * 16, 16)] = zvec
        return _
    lax.fori_loop(0, CH * LPR, _fill_zc, None)

    def _zero_agg(c, _):
        pltpu.sync_copy(zeroc, agg_sh.at[pl.ds(row0 + c * CH, CH)])
        return _
    lax.fori_loop(0, NCC, _zero_agg, None)
    plsc.subcore_barrier()

    def _scatter_phase(src_ref):
        pltpu.async_copy(src_ref.at[cid].at[src_v.at[0]], rowbuf, sem)
        pltpu.async_copy(src_ref.at[cid].at[src_v.at[1]], rowbuf1, sem1)

        def _grp(g, _):
            j0 = g * 2
            j1 = j0 + 1
            pltpu.make_async_copy(
                src_ref.at[cid].at[src_v.at[j0]], rowbuf, sem).wait()
            pltpu.async_copy(rowbuf, agg_sh.at[dst_v.at[j0]], ssem,
                             add=True)
            pltpu.make_async_copy(
                src_ref.at[cid].at[src_v.at[j1]], rowbuf1, sem1).wait()
            pltpu.async_copy(rowbuf1, agg_sh.at[dst_v.at[j1]], ssem1,
                             add=True)
            pltpu.make_async_copy(
                rowbuf, agg_sh.at[dst_v.at[j0]], ssem).wait()

            @pl.when(g < NCH // 2 - 1)
            def _():
                pltpu.async_copy(
                    src_ref.at[cid].at[src_v.at[j0 + 2]], rowbuf, sem)
            pltpu.make_async_copy(
                rowbuf1, agg_sh.at[dst_v.at[j1]], ssem1).wait()

            @pl.when(g < NCH // 2 - 1)
            def _():
                pltpu.async_copy(
                    src_ref.at[cid].at[src_v.at[j1 + 2]], rowbuf1, sem1)
            return _
        lax.fori_loop(0, NCH // 2, _grp, None)
        plsc.subcore_barrier()

    def _update_phase(last):
        def _upd_chunk(c, _):
            r0 = row0 + c * CH
            pltpu.sync_copy(agg_sh.at[pl.ds(r0, CH)], aggc)
            pltpu.sync_copy(zeroc, agg_sh.at[pl.ds(r0, CH)])

            def _rows(r, _):
                i = c * CH + r
                ns = nstri[i, pl.ds(0, 16)]
                nv = nstri[i, pl.ds(16, 16)]
                f = jnp.where(last, 1.0, ns)
                for v in range(LPR):
                    sl = pl.ds(v * 16, 16)
                    outc[r, sl] = f * (ahc[i, sl] + nv * aggc[r, sl])
                return _
            lax.fori_loop(0, CH, _rows, None)

            @pl.when(jnp.logical_not(last))
            def _():
                pltpu.sync_copy(outc, xs_hbm.at[cid, pl.ds(r0, CH)])

            @pl.when(last)
            def _():
                pltpu.sync_copy(outc, out_hbm.at[cid, pl.ds(r0, CH)])
            return _
        lax.fori_loop(0, NCC, _upd_chunk, None)
        plsc.subcore_barrier()

    _scatter_phase(xs0_hbm)
    _update_phase(jnp.bool_(False))

    def _iter(k, _):
        _scatter_phase(xs_hbm)
        _update_phase(k == K - 2)
        return _
    lax.fori_loop(0, K - 1, _iter, None)


def _sc_propagate(ahh, normc, srcp, dstp, xs0):
    mesh = plsc.VectorSubcoreMesh(core_axis_name="c", subcore_axis_name="s")
    fn = functools.partial(
        pl.kernel,
        mesh=mesh,
        compiler_params=pltpu.CompilerParams(use_tc_tiling_on_sc=False),
        out_type=[
            jax.ShapeDtypeStruct((2, NN, DC), jnp.float32),
            jax.ShapeDtypeStruct((2, NN, DC), jnp.float32),
        ],
        scratch_types=[
            pltpu.VMEM((NCH, CH), jnp.int32),
            pltpu.VMEM((NCH, CH), jnp.int32),
            pltpu.VMEM((CH, DC), jnp.float32),
            pltpu.VMEM((CH, DC), jnp.float32),
            pltpu.VMEM((CH, DC), jnp.float32),
            pltpu.VMEM((NPT, DC), jnp.float32),
            pltpu.VMEM((CH, DC), jnp.float32),
            pltpu.VMEM((CH, DC), jnp.float32),
            pltpu.VMEM((NPT, 32), jnp.float32),
            pltpu.VMEM_SHARED((NN, DC), jnp.float32),
            pltpu.SemaphoreType.DMA,
            pltpu.SemaphoreType.DMA,
            pltpu.SemaphoreType.DMA,
            pltpu.SemaphoreType.DMA,
        ],
    )(_sc_propagate_body)
    return fn(ahh, normc, srcp, dstp, xs0)


def kernel(features, edge_index, W):
    src = edge_index[0].astype(jnp.int32).reshape(NS, EPT)
    dst = edge_index[1].astype(jnp.int32).reshape(NS, EPT)
    pad = ((0, 0), (0, EPAD - EPT))
    srcp = jnp.pad(src, pad, constant_values=DUMMY).reshape(NS, NCH, CH)
    dstp = jnp.pad(dst, pad, constant_values=DUMMY).reshape(NS, NCH, CH)
    srcdst = jnp.stack([srcp, dstp])

    feat_pad = jnp.pad(features, ((0, NN - N_NODES), (0, 0)))
    hp, ah = _project(feat_pad, W.T)

    cnt = _sc_degrees(srcdst)
    normc, ahh, xs0 = _prep(cnt, ah, hp)

    out_pad, _ = _sc_propagate(ahh, normc, srcp, dstp, xs0)
    return jnp.concatenate([out_pad[0], out_pad[1]], axis=1)[:N_NODES]


# speedup vs baseline: 2.1874x; 1.2837x over previous
import functools

import jax
import jax.numpy as jnp
from jax import lax
from jax.experimental import pallas as pl
from jax.experimental.pallas import tpu as pltpu
from jax.experimental.pallas import tpu_sc as plsc

N_NODES = 10000
N_EDGES = 160000
D_FEAT = 256
N_CLASSES = 64
K = 10
ALPHA = 0.1

NS = 16
NN = 10240
NPT = NN // NS
CH = 128
EPT = N_EDGES // NS
NCH = 80
EPAD = NCH * CH
DUMMY = N_NODES
D = N_CLASSES
DC = D // 2
LPR = DC // 16
NCC = NPT // CH
NB = 4
OMA = 1.0 - ALPHA


def _matmul_body(x_ref, w_ref, h_ref, ah_ref):
    h = jnp.dot(x_ref[...], w_ref[...], preferred_element_type=jnp.float32)
    h_ref[...] = h
    ah_ref[...] = h * ALPHA


def _project(feat_pad, wt):
    blk = 1024
    return pl.pallas_call(
        _matmul_body,
        grid=(NN // blk,),
        in_specs=[
            pl.BlockSpec((blk, D_FEAT), lambda i: (i, 0)),
            pl.BlockSpec((D_FEAT, D), lambda i: (0, 0)),
        ],
        out_specs=[
            pl.BlockSpec((blk, D), lambda i: (i, 0)),
            pl.BlockSpec((blk, D), lambda i: (i, 0)),
        ],
        out_shape=[
            jax.ShapeDtypeStruct((NN, D), jnp.float32),
            jax.ShapeDtypeStruct((NN, D), jnp.float32),
        ],
    )(feat_pad, wt)


def _prep_body(cnt_ref, ah_ref, hp_ref, normc_ref, ahh_ref, xs0_ref):
    ns = lax.rsqrt(jnp.maximum(cnt_ref[0, :, :1], 1.0))
    nv = OMA * lax.rsqrt(jnp.maximum(cnt_ref[1, :, :1], 1.0))
    normc_ref[...] = jnp.concatenate(
        [jnp.broadcast_to(ns, (NN, 16)),
         jnp.broadcast_to(nv, (NN, 16))], axis=1)
    ah = ah_ref[...]
    ahh_ref[0] = ah[:, :DC]
    ahh_ref[1] = ah[:, DC:]
    xs0 = ns * hp_ref[...]
    xs0_ref[0] = xs0[:, :DC]
    xs0_ref[1] = xs0[:, DC:]


def _prep(cnt, ah, hp):
    return pl.pallas_call(
        _prep_body,
        out_shape=[
            jax.ShapeDtypeStruct((NN, 32), jnp.float32),
            jax.ShapeDtypeStruct((2, NN, DC), jnp.float32),
            jax.ShapeDtypeStruct((2, NN, DC), jnp.float32),
        ],
    )(cnt, ah, hp)


def _sc_degrees_body(sd_hbm, cnt_hbm, idx_v, ones16, zbuf16, cnt_sh):
    cid = lax.axis_index("c")
    tid = lax.axis_index("s")
    row0 = tid * NPT
    zvec = jnp.zeros((16,), jnp.float32)
    ovec = jnp.ones((16,), jnp.float32)

    pltpu.sync_copy(sd_hbm.at[cid, tid], idx_v)

    def _fill_o(i, _):
        ones16[i, pl.ds(0, 16)] = ovec
        return _
    lax.fori_loop(0, CH, _fill_o, None)

    def _fill_z16(i, _):
        zbuf16[i, pl.ds(0, 16)] = zvec
        return _
    lax.fori_loop(0, NPT, _fill_z16, None)

    pltpu.sync_copy(zbuf16, cnt_sh.at[pl.ds(row0, NPT)])
    plsc.subcore_barrier()

    def _deg_chunk(j, _):
        pltpu.sync_copy(ones16, cnt_sh.at[idx_v.at[j]], add=True)
        return _
    lax.fori_loop(0, NCH, _deg_chunk, None)
    plsc.subcore_barrier()

    pltpu.sync_copy(cnt_sh.at[pl.ds(row0, NPT)],
                    cnt_hbm.at[cid, pl.ds(row0, NPT)])


def _sc_degrees(srcdst):
    mesh = plsc.VectorSubcoreMesh(core_axis_name="c", subcore_axis_name="s")
    fn = functools.partial(
        pl.kernel,
        mesh=mesh,
        compiler_params=pltpu.CompilerParams(use_tc_tiling_on_sc=False),
        out_type=jax.ShapeDtypeStruct((2, NN, 16), jnp.float32),
        scratch_types=[
            pltpu.VMEM((NCH, CH), jnp.int32),
            pltpu.VMEM((CH, 16), jnp.float32),
            pltpu.VMEM((NPT, 16), jnp.float32),
            pltpu.VMEM_SHARED((NN, 16), jnp.float32),
        ],
    )(_sc_degrees_body)
    return fn(srcdst)


def _sc_propagate_body(ahh_hbm, normc_hbm, src_hbm, dst_hbm, xs0_hbm,
                       out_hbm, xs_hbm,
                       src_v, dst_v, rowbuf, rowbuf1, rowbuf2, rowbuf3,
                       aggc, ahc, outc, zeroc, nstri, agg_sh,
                       sem, sem1, sem2, sem3, ssem, ssem1, ssem2, ssem3):
    cid = lax.axis_index("c")
    tid = lax.axis_index("s")
    row0 = tid * NPT
    zvec = jnp.zeros((16,), jnp.float32)

    pltpu.sync_copy(src_hbm.at[tid], src_v)
    pltpu.sync_copy(dst_hbm.at[tid], dst_v)
    pltpu.sync_copy(normc_hbm.at[pl.ds(row0, NPT)], nstri)
    pltpu.sync_copy(ahh_hbm.at[cid, pl.ds(row0, NPT)], ahc)

    def _fill_zc(i, _):
        zeroc[i // LPR, pl.ds((i % LPR) * 16, 16)] = zvec
        return _
    lax.fori_loop(0, CH * LPR, _fill_zc, None)

    def _zero_agg(c, _):
        pltpu.sync_copy(zeroc, agg_sh.at[pl.ds(row0 + c * CH, CH)])
        return _
    lax.fori_loop(0, NCC, _zero_agg, None)
    plsc.subcore_barrier()

    rbufs = [rowbuf, rowbuf1, rowbuf2, rowbuf3]
    gsems = [sem, sem1, sem2, sem3]
    ssems = [ssem, ssem1, ssem2, ssem3]

    def _scatter_phase(src_ref):
        for b in range(NB):
            pltpu.async_copy(src_ref.at[cid].at[src_v.at[b]],
                             rbufs[b], gsems[b])

        def _grp(g, _):
            for b in range(NB):
                j = g * NB + b
                pltpu.make_async_copy(
                    src_ref.at[cid].at[src_v.at[j]],
                    rbufs[b], gsems[b]).wait()
                pltpu.async_copy(rbufs[b], agg_sh.at[dst_v.at[j]],
                                 ssems[b], add=True)
            for b in range(NB):
                j = g * NB + b
                pltpu.make_async_copy(
                    rbufs[b], agg_sh.at[dst_v.at[j]], ssems[b]).wait()

                @pl.when(g < NCH // NB - 1)
                def _():
                    pltpu.async_copy(
                        src_ref.at[cid].at[src_v.at[j + NB]],
                        rbufs[b], gsems[b])
            return _
        lax.fori_loop(0, NCH // NB, _grp, None)
        plsc.subcore_barrier()

    def _update_phase(last):
        def _upd_chunk(c, _):
            r0 = row0 + c * CH
            pltpu.sync_copy(agg_sh.at[pl.ds(r0, CH)], aggc)
            pltpu.sync_copy(zeroc, agg_sh.at[pl.ds(r0, CH)])

            def _rows(r, _):
                i = c * CH + r
                ns = nstri[i, pl.ds(0, 16)]
                nv = nstri[i, pl.ds(16, 16)]
                f = jnp.where(last, 1.0, ns)
                for v in range(LPR):
                    sl = pl.ds(v * 16, 16)
                    outc[r, sl] = f * (ahc[i, sl] + nv * aggc[r, sl])
                return _
            lax.fori_loop(0, CH, _rows, None)

            @pl.when(jnp.logical_not(last))
            def _():
                pltpu.sync_copy(outc, xs_hbm.at[cid, pl.ds(r0, CH)])

            @pl.when(last)
            def _():
                pltpu.sync_copy(outc, out_hbm.at[cid, pl.ds(r0, CH)])
            return _
        lax.fori_loop(0, NCC, _upd_chunk, None)
        plsc.subcore_barrier()

    _scatter_phase(xs0_hbm)
    _update_phase(jnp.bool_(False))

    def _iter(k, _):
        _scatter_phase(xs_hbm)
        _update_phase(k == K - 2)
        return _
    lax.fori_loop(0, K - 1, _iter, None)


def _sc_propagate(ahh, normc, srcp, dstp, xs0):
    mesh = plsc.VectorSubcoreMesh(core_axis_name="c", subcore_axis_name="s")
    fn = functools.partial(
        pl.kernel,
        mesh=mesh,
        compiler_params=pltpu.CompilerParams(use_tc_tiling_on_sc=False),
        out_type=[
            jax.ShapeDtypeStruct((2, NN, DC), jnp.float32),
            jax.ShapeDtypeStruct((2, NN, DC), jnp.float32),
        ],
        scratch_types=[
            pltpu.VMEM((NCH, CH), jnp.int32),
            pltpu.VMEM((NCH, CH), jnp.int32),
            pltpu.VMEM((CH, DC), jnp.float32),
            pltpu.VMEM((CH, DC), jnp.float32),
            pltpu.VMEM((CH, DC), jnp.float32),
            pltpu.VMEM((CH, DC), jnp.float32),
            pltpu.VMEM((CH, DC), jnp.float32),
            pltpu.VMEM((NPT, DC), jnp.float32),
            pltpu.VMEM((CH, DC), jnp.float32),
            pltpu.VMEM((CH, DC), jnp.float32),
            pltpu.VMEM((NPT, 32), jnp.float32),
            pltpu.VMEM_SHARED((NN, DC), jnp.float32),
            pltpu.SemaphoreType.DMA,
            pltpu.SemaphoreType.DMA,
            pltpu.SemaphoreType.DMA,
            pltpu.SemaphoreType.DMA,
            pltpu.SemaphoreType.DMA,
            pltpu.SemaphoreType.DMA,
            pltpu.SemaphoreType.DMA,
            pltpu.SemaphoreType.DMA,
        ],
    )(_sc_propagate_body)
    return fn(ahh, normc, srcp, dstp, xs0)


def kernel(features, edge_index, W):
    src = edge_index[0].astype(jnp.int32).reshape(NS, EPT)
    dst = edge_index[1].astype(jnp.int32).reshape(NS, EPT)
    pad = ((0, 0), (0, EPAD - EPT))
    srcp = jnp.pad(src, pad, constant_values=DUMMY).reshape(NS, NCH, CH)
    dstp = jnp.pad(dst, pad, constant_values=DUMMY).reshape(NS, NCH, CH)
    srcdst = jnp.stack([srcp, dstp])

    feat_pad = jnp.pad(features, ((0, NN - N_NODES), (0, 0)))
    hp, ah = _project(feat_pad, W.T)

    cnt = _sc_degrees(srcdst)
    normc, ahh, xs0 = _prep(cnt, ah, hp)

    out_pad, _ = _sc_propagate(ahh, normc, srcp, dstp, xs0)
    return jnp.concatenate([out_pad[0], out_pad[1]], axis=1)[:N_NODES]


# speedup vs baseline: 2.3067x; 1.0545x over previous
import functools

import jax
import jax.numpy as jnp
from jax import lax
from jax.experimental import pallas as pl
from jax.experimental.pallas import tpu as pltpu
from jax.experimental.pallas import tpu_sc as plsc

N_NODES = 10000
N_EDGES = 160000
D_FEAT = 256
N_CLASSES = 64
K = 10
ALPHA = 0.1

NS = 16
NN = 10240
NPT = NN // NS
CH = 128
EPT = N_EDGES // NS
NCH = 80
EPAD = NCH * CH
DUMMY = N_NODES
D = N_CLASSES
DC = D // 2
LPR = DC // 16
NCC = NPT // CH
NB = 8
OMA = 1.0 - ALPHA


def _matmul_body(x_ref, w_ref, h_ref, ah_ref):
    h = jnp.dot(x_ref[...], w_ref[...], preferred_element_type=jnp.float32)
    h_ref[...] = h
    ah_ref[...] = h * ALPHA


def _project(feat_pad, wt):
    blk = 1024
    return pl.pallas_call(
        _matmul_body,
        grid=(NN // blk,),
        in_specs=[
            pl.BlockSpec((blk, D_FEAT), lambda i: (i, 0)),
            pl.BlockSpec((D_FEAT, D), lambda i: (0, 0)),
        ],
        out_specs=[
            pl.BlockSpec((blk, D), lambda i: (i, 0)),
            pl.BlockSpec((blk, D), lambda i: (i, 0)),
        ],
        out_shape=[
            jax.ShapeDtypeStruct((NN, D), jnp.float32),
            jax.ShapeDtypeStruct((NN, D), jnp.float32),
        ],
    )(feat_pad, wt)


def _prep_body(cnt_ref, ah_ref, hp_ref, normc_ref, ahh_ref, xs0_ref):
    ns = lax.rsqrt(jnp.maximum(cnt_ref[0, :, :1], 1.0))
    nv = OMA * lax.rsqrt(jnp.maximum(cnt_ref[1, :, :1], 1.0))
    normc_ref[...] = jnp.concatenate(
        [jnp.broadcast_to(ns, (NN, 16)),
         jnp.broadcast_to(nv, (NN, 16))], axis=1)
    ah = ah_ref[...]
    ahh_ref[0] = ah[:, :DC]
    ahh_ref[1] = ah[:, DC:]
    xs0 = ns * hp_ref[...]
    xs0_ref[0] = xs0[:, :DC]
    xs0_ref[1] = xs0[:, DC:]


def _prep(cnt, ah, hp):
    return pl.pallas_call(
        _prep_body,
        out_shape=[
            jax.ShapeDtypeStruct((NN, 32), jnp.float32),
            jax.ShapeDtypeStruct((2, NN, DC), jnp.float32),
            jax.ShapeDtypeStruct((2, NN, DC), jnp.float32),
        ],
    )(cnt, ah, hp)


def _sc_degrees_body(sd_hbm, cnt_hbm, idx_v, ones16, zbuf16, cnt_sh):
    cid = lax.axis_index("c")
    tid = lax.axis_index("s")
    row0 = tid * NPT
    zvec = jnp.zeros((16,), jnp.float32)
    ovec = jnp.ones((16,), jnp.float32)

    pltpu.sync_copy(sd_hbm.at[cid, tid], idx_v)

    def _fill_o(i, _):
        ones16[i, pl.ds(0, 16)] = ovec
        return _
    lax.fori_loop(0, CH, _fill_o, None)

    def _fill_z16(i, _):
        zbuf16[i, pl.ds(0, 16)] = zvec
        return _
    lax.fori_loop(0, NPT, _fill_z16, None)

    pltpu.sync_copy(zbuf16, cnt_sh.at[pl.ds(row0, NPT)])
    plsc.subcore_barrier()

    def _deg_chunk(j, _):
        pltpu.sync_copy(ones16, cnt_sh.at[idx_v.at[j]], add=True)
        return _
    lax.fori_loop(0, NCH, _deg_chunk, None)
    plsc.subcore_barrier()

    pltpu.sync_copy(cnt_sh.at[pl.ds(row0, NPT)],
                    cnt_hbm.at[cid, pl.ds(row0, NPT)])


def _sc_degrees(srcdst):
    mesh = plsc.VectorSubcoreMesh(core_axis_name="c", subcore_axis_name="s")
    fn = functools.partial(
        pl.kernel,
        mesh=mesh,
        compiler_params=pltpu.CompilerParams(use_tc_tiling_on_sc=False),
        out_type=jax.ShapeDtypeStruct((2, NN, 16), jnp.float32),
        scratch_types=[
            pltpu.VMEM((NCH, CH), jnp.int32),
            pltpu.VMEM((CH, 16), jnp.float32),
            pltpu.VMEM((NPT, 16), jnp.float32),
            pltpu.VMEM_SHARED((NN, 16), jnp.float32),
        ],
    )(_sc_degrees_body)
    return fn(srcdst)


def _sc_propagate_body(ahh_hbm, normc_hbm, src_hbm, dst_hbm, xs0_hbm,
                       out_hbm, xs_hbm,
                       src_v, dst_v, rowbuf, rowbuf1, rowbuf2, rowbuf3,
                       rowbuf4, rowbuf5, rowbuf6, rowbuf7,
                       aggc, ahc, outc, zeroc, nstri, agg_sh,
                       sem, sem1, sem2, sem3, sem4, sem5, sem6, sem7,
                       ssem, ssem1, ssem2, ssem3,
                       ssem4, ssem5, ssem6, ssem7):
    cid = lax.axis_index("c")
    tid = lax.axis_index("s")
    row0 = tid * NPT
    zvec = jnp.zeros((16,), jnp.float32)

    pltpu.sync_copy(src_hbm.at[tid], src_v)
    pltpu.sync_copy(dst_hbm.at[tid], dst_v)
    pltpu.sync_copy(normc_hbm.at[pl.ds(row0, NPT)], nstri)
    pltpu.sync_copy(ahh_hbm.at[cid, pl.ds(row0, NPT)], ahc)

    def _fill_zc(i, _):
        zeroc[i // LPR, pl.ds((i % LPR) * 16, 16)] = zvec
        return _
    lax.fori_loop(0, CH * LPR, _fill_zc, None)

    def _zero_agg(c, _):
        pltpu.sync_copy(zeroc, agg_sh.at[pl.ds(row0 + c * CH, CH)])
        return _
    lax.fori_loop(0, NCC, _zero_agg, None)
    plsc.subcore_barrier()

    rbufs = [rowbuf, rowbuf1, rowbuf2, rowbuf3,
             rowbuf4, rowbuf5, rowbuf6, rowbuf7]
    gsems = [sem, sem1, sem2, sem3, sem4, sem5, sem6, sem7]
    ssems = [ssem, ssem1, ssem2, ssem3, ssem4, ssem5, ssem6, ssem7]

    def _scatter_phase(src_ref):
        for b in range(NB):
            pltpu.async_copy(src_ref.at[cid].at[src_v.at[b]],
                             rbufs[b], gsems[b])

        def _grp(g, _):
            for b in range(NB):
                j = g * NB + b
                pltpu.make_async_copy(
                    src_ref.at[cid].at[src_v.at[j]],
                    rbufs[b], gsems[b]).wait()
                pltpu.async_copy(rbufs[b], agg_sh.at[dst_v.at[j]],
                                 ssems[b], add=True)
            for b in range(NB):
                j = g * NB + b
                pltpu.make_async_copy(
                    rbufs[b], agg_sh.at[dst_v.at[j]], ssems[b]).wait()

                @pl.when(g < NCH // NB - 1)
                def _():
                    pltpu.async_copy(
                        src_ref.at[cid].at[src_v.at[j + NB]],
                        rbufs[b], gsems[b])
            return _
        lax.fori_loop(0, NCH // NB, _grp, None)
        plsc.subcore_barrier()

    def _update_phase(last):
        def _upd_chunk(c, _):
            r0 = row0 + c * CH
            pltpu.sync_copy(agg_sh.at[pl.ds(r0, CH)], aggc)
            pltpu.sync_copy(zeroc, agg_sh.at[pl.ds(r0, CH)])

            def _rows(r, _):
                i = c * CH + r
                ns = nstri[i, pl.ds(0, 16)]
                nv = nstri[i, pl.ds(16, 16)]
                f = jnp.where(last, 1.0, ns)
                for v in range(LPR):
                    sl = pl.ds(v * 16, 16)
                    outc[r, sl] = f * (ahc[i, sl] + nv * aggc[r, sl])
                return _
            lax.fori_loop(0, CH, _rows, None)

            @pl.when(jnp.logical_not(last))
            def _():
                pltpu.sync_copy(outc, xs_hbm.at[cid, pl.ds(r0, CH)])

            @pl.when(last)
            def _():
                pltpu.sync_copy(outc, out_hbm.at[cid, pl.ds(r0, CH)])
            return _
        lax.fori_loop(0, NCC, _upd_chunk, None)
        plsc.subcore_barrier()

    _scatter_phase(xs0_hbm)
    _update_phase(jnp.bool_(False))

    def _iter(k, _):
        _scatter_phase(xs_hbm)
        _update_phase(k == K - 2)
        return _
    lax.fori_loop(0, K - 1, _iter, None)


def _sc_propagate(ahh, normc, srcp, dstp, xs0):
    mesh = plsc.VectorSubcoreMesh(core_axis_name="c", subcore_axis_name="s")
    fn = functools.partial(
        pl.kernel,
        mesh=mesh,
        compiler_params=pltpu.CompilerParams(use_tc_tiling_on_sc=False),
        out_type=[
            jax.ShapeDtypeStruct((2, NN, DC), jnp.float32),
            jax.ShapeDtypeStruct((2, NN, DC), jnp.float32),
        ],
        scratch_types=[
            pltpu.VMEM((NCH, CH), jnp.int32),
            pltpu.VMEM((NCH, CH), jnp.int32),
            pltpu.VMEM((CH, DC), jnp.float32),
            pltpu.VMEM((CH, DC), jnp.float32),
            pltpu.VMEM((CH, DC), jnp.float32),
            pltpu.VMEM((CH, DC), jnp.float32),
            pltpu.VMEM((CH, DC), jnp.float32),
            pltpu.VMEM((CH, DC), jnp.float32),
            pltpu.VMEM((CH, DC), jnp.float32),
            pltpu.VMEM((CH, DC), jnp.float32),
            pltpu.VMEM((CH, DC), jnp.float32),
            pltpu.VMEM((NPT, DC), jnp.float32),
            pltpu.VMEM((CH, DC), jnp.float32),
            pltpu.VMEM((CH, DC), jnp.float32),
            pltpu.VMEM((NPT, 32), jnp.float32),
            pltpu.VMEM_SHARED((NN, DC), jnp.float32),
            pltpu.SemaphoreType.DMA,
            pltpu.SemaphoreType.DMA,
            pltpu.SemaphoreType.DMA,
            pltpu.SemaphoreType.DMA,
            pltpu.SemaphoreType.DMA,
            pltpu.SemaphoreType.DMA,
            pltpu.SemaphoreType.DMA,
            pltpu.SemaphoreType.DMA,
            pltpu.SemaphoreType.DMA,
            pltpu.SemaphoreType.DMA,
            pltpu.SemaphoreType.DMA,
            pltpu.SemaphoreType.DMA,
            pltpu.SemaphoreType.DMA,
            pltpu.SemaphoreType.DMA,
            pltpu.SemaphoreType.DMA,
            pltpu.SemaphoreType.DMA,
        ],
    )(_sc_propagate_body)
    return fn(ahh, normc, srcp, dstp, xs0)


def kernel(features, edge_index, W):
    src = edge_index[0].astype(jnp.int32).reshape(NS, EPT)
    dst = edge_index[1].astype(jnp.int32).reshape(NS, EPT)
    pad = ((0, 0), (0, EPAD - EPT))
    srcp = jnp.pad(src, pad, constant_values=DUMMY).reshape(NS, NCH, CH)
    dstp = jnp.pad(dst, pad, constant_values=DUMMY).reshape(NS, NCH, CH)
    srcdst = jnp.stack([srcp, dstp])

    feat_pad = jnp.pad(features, ((0, NN - N_NODES), (0, 0)))
    hp, ah = _project(feat_pad, W.T)

    cnt = _sc_degrees(srcdst)
    normc, ahh, xs0 = _prep(cnt, ah, hp)

    out_pad, _ = _sc_propagate(ahh, normc, srcp, dstp, xs0)
    return jnp.concatenate([out_pad[0], out_pad[1]], axis=1)[:N_NODES]


# speedup vs baseline: 2.3318x; 1.0109x over previous
import functools

import jax
import jax.numpy as jnp
from jax import lax
from jax.experimental import pallas as pl
from jax.experimental.pallas import tpu as pltpu
from jax.experimental.pallas import tpu_sc as plsc

N_NODES = 10000
N_EDGES = 160000
D_FEAT = 256
N_CLASSES = 64
K = 10
ALPHA = 0.1

NS = 16
NN = 10240
NPT = NN // NS
CH = 128
EPT = N_EDGES // NS
NCH = 80
EPAD = NCH * CH
DUMMY = N_NODES
D = N_CLASSES
DC = D // 2
LPR = DC // 16
NCC = NPT // CH
NB = 8
OMA = 1.0 - ALPHA


def _matmul_body(x_ref, w_ref, h_ref, ah_ref):
    h = jnp.dot(x_ref[...], w_ref[...], preferred_element_type=jnp.float32)
    h_ref[...] = h
    ah_ref[...] = h * ALPHA


def _project(feat_pad, wt):
    blk = 1024
    return pl.pallas_call(
        _matmul_body,
        grid=(NN // blk,),
        in_specs=[
            pl.BlockSpec((blk, D_FEAT), lambda i: (i, 0)),
            pl.BlockSpec((D_FEAT, D), lambda i: (0, 0)),
        ],
        out_specs=[
            pl.BlockSpec((blk, D), lambda i: (i, 0)),
            pl.BlockSpec((blk, D), lambda i: (i, 0)),
        ],
        out_shape=[
            jax.ShapeDtypeStruct((NN, D), jnp.float32),
            jax.ShapeDtypeStruct((NN, D), jnp.float32),
        ],
    )(feat_pad, wt)


def _prep_body(cnt_ref, ah_ref, hp_ref, normc_ref, ahh_ref, xs0_ref):
    ns = lax.rsqrt(jnp.maximum(cnt_ref[0, :, :1], 1.0))
    nv = OMA * lax.rsqrt(jnp.maximum(cnt_ref[1, :, :1], 1.0))
    normc_ref[...] = jnp.concatenate(
        [jnp.broadcast_to(ns, (NN, 16)),
         jnp.broadcast_to(nv, (NN, 16))], axis=1)
    ah = ah_ref[...]
    ahh_ref[0] = ah[:, :DC]
    ahh_ref[1] = ah[:, DC:]
    xs0 = ns * hp_ref[...]
    xs0_ref[0] = xs0[:, :DC]
    xs0_ref[1] = xs0[:, DC:]


def _prep(cnt, ah, hp):
    return pl.pallas_call(
        _prep_body,
        out_shape=[
            jax.ShapeDtypeStruct((NN, 32), jnp.float32),
            jax.ShapeDtypeStruct((2, NN, DC), jnp.float32),
            jax.ShapeDtypeStruct((2, NN, DC), jnp.float32),
        ],
    )(cnt, ah, hp)


def _sc_degrees_body(sd_hbm, cnt_hbm, idx_v, ones16, zbuf16, cnt_sh):
    cid = lax.axis_index("c")
    tid = lax.axis_index("s")
    row0 = tid * NPT
    zvec = jnp.zeros((16,), jnp.float32)
    ovec = jnp.ones((16,), jnp.float32)

    pltpu.sync_copy(sd_hbm.at[cid, tid], idx_v)

    def _fill_o(i, _):
        ones16[i, pl.ds(0, 16)] = ovec
        return _
    lax.fori_loop(0, CH, _fill_o, None)

    def _fill_z16(i, _):
        zbuf16[i, pl.ds(0, 16)] = zvec
        return _
    lax.fori_loop(0, NPT, _fill_z16, None)

    pltpu.sync_copy(zbuf16, cnt_sh.at[pl.ds(row0, NPT)])
    plsc.subcore_barrier()

    def _deg_chunk(j, _):
        pltpu.sync_copy(ones16, cnt_sh.at[idx_v.at[j]], add=True)
        return _
    lax.fori_loop(0, NCH, _deg_chunk, None)
    plsc.subcore_barrier()

    pltpu.sync_copy(cnt_sh.at[pl.ds(row0, NPT)],
                    cnt_hbm.at[cid, pl.ds(row0, NPT)])


def _sc_degrees(srcdst):
    mesh = plsc.VectorSubcoreMesh(core_axis_name="c", subcore_axis_name="s")
    fn = functools.partial(
        pl.kernel,
        mesh=mesh,
        compiler_params=pltpu.CompilerParams(use_tc_tiling_on_sc=False),
        out_type=jax.ShapeDtypeStruct((2, NN, 16), jnp.float32),
        scratch_types=[
            pltpu.VMEM((NCH, CH), jnp.int32),
            pltpu.VMEM((CH, 16), jnp.float32),
            pltpu.VMEM((NPT, 16), jnp.float32),
            pltpu.VMEM_SHARED((NN, 16), jnp.float32),
        ],
    )(_sc_degrees_body)
    return fn(srcdst)


def _sc_propagate_body(ahh_hbm, normc_hbm, src_hbm, dst_hbm, xs0_hbm,
                       out_hbm, xs_hbm,
                       src_v, dst_v, rowbuf, rowbuf1, rowbuf2, rowbuf3,
                       rowbuf4, rowbuf5, rowbuf6, rowbuf7,
                       ahc, zeroc, nstri, agg_sh,
                       sem, sem1, sem2, sem3, sem4, sem5, sem6, sem7,
                       ssem, ssem1, ssem2, ssem3,
                       ssem4, ssem5, ssem6, ssem7):
    cid = lax.axis_index("c")
    tid = lax.axis_index("s")
    row0 = tid * NPT
    zvec = jnp.zeros((16,), jnp.float32)

    pltpu.sync_copy(src_hbm.at[tid], src_v)
    pltpu.sync_copy(dst_hbm.at[tid], dst_v)
    pltpu.sync_copy(normc_hbm.at[pl.ds(row0, NPT)], nstri)
    pltpu.sync_copy(ahh_hbm.at[cid, pl.ds(row0, NPT)], ahc)

    def _fill_zc(i, _):
        zeroc[i // LPR, pl.ds((i % LPR) * 16, 16)] = zvec
        return _
    lax.fori_loop(0, CH * LPR, _fill_zc, None)

    def _zero_agg(c, _):
        pltpu.sync_copy(zeroc, agg_sh.at[pl.ds(row0 + c * CH, CH)])
        return _
    lax.fori_loop(0, NCC, _zero_agg, None)
    plsc.subcore_barrier()

    rbufs = [rowbuf, rowbuf1, rowbuf2, rowbuf3,
             rowbuf4, rowbuf5, rowbuf6, rowbuf7]
    gsems = [sem, sem1, sem2, sem3, sem4, sem5, sem6, sem7]
    ssems = [ssem, ssem1, ssem2, ssem3, ssem4, ssem5, ssem6, ssem7]

    def _scatter_phase(src_ref):
        for b in range(NB):
            pltpu.async_copy(src_ref.at[cid].at[src_v.at[b]],
                             rbufs[b], gsems[b])

        def _grp(g, _):
            for b in range(NB):
                j = g * NB + b
                pltpu.make_async_copy(
                    src_ref.at[cid].at[src_v.at[j]],
                    rbufs[b], gsems[b]).wait()
                pltpu.async_copy(rbufs[b], agg_sh.at[dst_v.at[j]],
                                 ssems[b], add=True)
            for b in range(NB):
                j = g * NB + b
                pltpu.make_async_copy(
                    rbufs[b], agg_sh.at[dst_v.at[j]], ssems[b]).wait()

                @pl.when(g < NCH // NB - 1)
                def _():
                    pltpu.async_copy(
                        src_ref.at[cid].at[src_v.at[j + NB]],
                        rbufs[b], gsems[b])
            return _
        lax.fori_loop(0, NCH // NB, _grp, None)
        plsc.subcore_barrier()

    def _update_phase(last):
        def _upd_chunk(c, _):
            r0 = row0 + c * CH
            ac = rbufs[0]
            oc = rbufs[1]
            pltpu.sync_copy(agg_sh.at[pl.ds(r0, CH)], ac)
            pltpu.async_copy(zeroc, agg_sh.at[pl.ds(r0, CH)], ssems[0])

            def _rows(r, _):
                i = c * CH + r
                ns = nstri[i, pl.ds(0, 16)]
                nv = nstri[i, pl.ds(16, 16)]
                f = jnp.where(last, 1.0, ns)
                for v in range(LPR):
                    sl = pl.ds(v * 16, 16)
                    oc[r, sl] = f * (ahc[i, sl] + nv * ac[r, sl])
                return _
            lax.fori_loop(0, CH, _rows, None)

            @pl.when(jnp.logical_not(last))
            def _():
                pltpu.sync_copy(oc, xs_hbm.at[cid, pl.ds(r0, CH)])

            @pl.when(last)
            def _():
                pltpu.sync_copy(oc, out_hbm.at[cid, pl.ds(r0, CH)])
            pltpu.make_async_copy(
                zeroc, agg_sh.at[pl.ds(r0, CH)], ssems[0]).wait()
            return _
        lax.fori_loop(0, NCC, _upd_chunk, None)
        plsc.subcore_barrier()

    _scatter_phase(xs0_hbm)
    _update_phase(jnp.bool_(False))

    def _iter(k, _):
        _scatter_phase(xs_hbm)
        _update_phase(k == K - 2)
        return _
    lax.fori_loop(0, K - 1, _iter, None)


def _sc_propagate(ahh, normc, srcp, dstp, xs0):
    mesh = plsc.VectorSubcoreMesh(core_axis_name="c", subcore_axis_name="s")
    fn = functools.partial(
        pl.kernel,
        mesh=mesh,
        compiler_params=pltpu.CompilerParams(use_tc_tiling_on_sc=False),
        out_type=[
            jax.ShapeDtypeStruct((2, NN, DC), jnp.float32),
            jax.ShapeDtypeStruct((2, NN, DC), jnp.float32),
        ],
        scratch_types=[
            pltpu.VMEM((NCH, CH), jnp.int32),
            pltpu.VMEM((NCH, CH), jnp.int32),
            pltpu.VMEM((CH, DC), jnp.float32),
            pltpu.VMEM((CH, DC), jnp.float32),
            pltpu.VMEM((CH, DC), jnp.float32),
            pltpu.VMEM((CH, DC), jnp.float32),
            pltpu.VMEM((CH, DC), jnp.float32),
            pltpu.VMEM((CH, DC), jnp.float32),
            pltpu.VMEM((CH, DC), jnp.float32),
            pltpu.VMEM((CH, DC), jnp.float32),
            pltpu.VMEM((NPT, DC), jnp.float32),
            pltpu.VMEM((CH, DC), jnp.float32),
            pltpu.VMEM((NPT, 32), jnp.float32),
            pltpu.VMEM_SHARED((NN, DC), jnp.float32),
            pltpu.SemaphoreType.DMA,
            pltpu.SemaphoreType.DMA,
            pltpu.SemaphoreType.DMA,
            pltpu.SemaphoreType.DMA,
            pltpu.SemaphoreType.DMA,
            pltpu.SemaphoreType.DMA,
            pltpu.SemaphoreType.DMA,
            pltpu.SemaphoreType.DMA,
            pltpu.SemaphoreType.DMA,
            pltpu.SemaphoreType.DMA,
            pltpu.SemaphoreType.DMA,
            pltpu.SemaphoreType.DMA,
            pltpu.SemaphoreType.DMA,
            pltpu.SemaphoreType.DMA,
            pltpu.SemaphoreType.DMA,
            pltpu.SemaphoreType.DMA,
        ],
    )(_sc_propagate_body)
    return fn(ahh, normc, srcp, dstp, xs0)


def kernel(features, edge_index, W):
    src = edge_index[0].astype(jnp.int32).reshape(NS, EPT)
    dst = edge_index[1].astype(jnp.int32).reshape(NS, EPT)
    pad = ((0, 0), (0, EPAD - EPT))
    srcp = jnp.pad(src, pad, constant_values=DUMMY).reshape(NS, NCH, CH)
    dstp = jnp.pad(dst, pad, constant_values=DUMMY).reshape(NS, NCH, CH)
    srcdst = jnp.stack([srcp, dstp])

    feat_pad = jnp.pad(features, ((0, NN - N_NODES), (0, 0)))
    hp, ah = _project(feat_pad, W.T)

    cnt = _sc_degrees(srcdst)
    normc, ahh, xs0 = _prep(cnt, ah, hp)

    out_pad, _ = _sc_propagate(ahh, normc, srcp, dstp, xs0)
    return jnp.concatenate([out_pad[0], out_pad[1]], axis=1)[:N_NODES]


# speedup vs baseline: 2.5973x; 1.1139x over previous
import functools

import jax
import jax.numpy as jnp
from jax import lax
from jax.experimental import pallas as pl
from jax.experimental.pallas import tpu as pltpu
from jax.experimental.pallas import tpu_sc as plsc

N_NODES = 10000
N_EDGES = 160000
D_FEAT = 256
N_CLASSES = 64
K = 10
ALPHA = 0.1

NS = 16
NN = 10240
NPT = NN // NS
CH = 128
EPT = N_EDGES // NS
NCH = 80
EPAD = NCH * CH
DUMMY = N_NODES
D = N_CLASSES
DC = D // 2
LPR = DC // 16
NCC = NPT // CH
NB = 8
OMA = 1.0 - ALPHA


def _matmul_body(x_ref, w_ref, h_ref, ah_ref):
    h = jnp.dot(x_ref[...], w_ref[...], preferred_element_type=jnp.float32)
    h_ref[...] = h
    ah_ref[...] = h * ALPHA


def _project(feat_pad, wt):
    blk = 1024
    return pl.pallas_call(
        _matmul_body,
        grid=(NN // blk,),
        in_specs=[
            pl.BlockSpec((blk, D_FEAT), lambda i: (i, 0)),
            pl.BlockSpec((D_FEAT, D), lambda i: (0, 0)),
        ],
        out_specs=[
            pl.BlockSpec((blk, D), lambda i: (i, 0)),
            pl.BlockSpec((blk, D), lambda i: (i, 0)),
        ],
        out_shape=[
            jax.ShapeDtypeStruct((NN, D), jnp.float32),
            jax.ShapeDtypeStruct((NN, D), jnp.float32),
        ],
    )(feat_pad, wt)


def _prep_body(cnt_ref, ah_ref, hp_ref, normc_ref, ahh_ref, xs0_ref):
    ns = lax.rsqrt(jnp.maximum(cnt_ref[0, :, :1], 1.0))
    nv = OMA * lax.rsqrt(jnp.maximum(cnt_ref[1, :, :1], 1.0))
    normc_ref[...] = jnp.concatenate(
        [jnp.broadcast_to(ns, (NN, 16)),
         jnp.broadcast_to(nv, (NN, 16))], axis=1)
    ah = ah_ref[...]
    ahh_ref[0] = ah[:, :DC]
    ahh_ref[1] = ah[:, DC:]
    xs0 = ns * hp_ref[...]
    xs0_ref[0] = xs0[:, :DC]
    xs0_ref[1] = xs0[:, DC:]


def _prep(cnt, ah, hp):
    return pl.pallas_call(
        _prep_body,
        out_shape=[
            jax.ShapeDtypeStruct((NN, 32), jnp.float32),
            jax.ShapeDtypeStruct((2, NN, DC), jnp.float32),
            jax.ShapeDtypeStruct((2, NN, DC), jnp.float32),
        ],
    )(cnt, ah, hp)


def _sc_degrees_body(sd_hbm, cnt_hbm, idx_v, ones16, zbuf16, cnt_sh):
    cid = lax.axis_index("c")
    tid = lax.axis_index("s")
    row0 = tid * NPT
    zvec = jnp.zeros((16,), jnp.float32)
    ovec = jnp.ones((16,), jnp.float32)

    pltpu.sync_copy(sd_hbm.at[cid, tid], idx_v)

    def _fill_o(i, _):
        ones16[i, pl.ds(0, 16)] = ovec
        return _
    lax.fori_loop(0, CH, _fill_o, None)

    def _fill_z16(i, _):
        zbuf16[i, pl.ds(0, 16)] = zvec
        return _
    lax.fori_loop(0, NPT, _fill_z16, None)

    pltpu.sync_copy(zbuf16, cnt_sh.at[pl.ds(row0, NPT)])
    plsc.subcore_barrier()

    def _deg_chunk(j, _):
        pltpu.sync_copy(ones16, cnt_sh.at[idx_v.at[j]], add=True)
        return _
    lax.fori_loop(0, NCH, _deg_chunk, None)
    plsc.subcore_barrier()

    pltpu.sync_copy(cnt_sh.at[pl.ds(row0, NPT)],
                    cnt_hbm.at[cid, pl.ds(row0, NPT)])


def _sc_degrees(srcdst):
    mesh = plsc.VectorSubcoreMesh(core_axis_name="c", subcore_axis_name="s")
    fn = functools.partial(
        pl.kernel,
        mesh=mesh,
        compiler_params=pltpu.CompilerParams(use_tc_tiling_on_sc=False),
        out_type=jax.ShapeDtypeStruct((2, NN, 16), jnp.float32),
        scratch_types=[
            pltpu.VMEM((NCH, CH), jnp.int32),
            pltpu.VMEM((CH, 16), jnp.float32),
            pltpu.VMEM((NPT, 16), jnp.float32),
            pltpu.VMEM_SHARED((NN, 16), jnp.float32),
        ],
    )(_sc_degrees_body)
    return fn(srcdst)


def _sc_propagate_body(ahh_hbm, normc_hbm, src_hbm, dst_hbm, xs0_hbm,
                       out_hbm, xs_hbm,
                       src_v, dst_v, rowbuf, rowbuf1, rowbuf2, rowbuf3,
                       rowbuf4, rowbuf5, rowbuf6, rowbuf7,
                       ahc, zeroc, nstri, agg_sh,
                       sem, sem1, sem2, sem3, sem4, sem5, sem6, sem7,
                       ssem, ssem1, ssem2, ssem3,
                       ssem4, ssem5, ssem6, ssem7):
    cid = lax.axis_index("c")
    tid = lax.axis_index("s")
    row0 = tid * NPT
    zvec = jnp.zeros((16,), jnp.float32)

    pltpu.sync_copy(src_hbm.at[tid], src_v)
    pltpu.sync_copy(dst_hbm.at[tid], dst_v)
    pltpu.sync_copy(normc_hbm.at[pl.ds(row0, NPT)], nstri)
    pltpu.sync_copy(ahh_hbm.at[cid, pl.ds(row0, NPT)], ahc)

    def _fill_zc(i, _):
        zeroc[i // LPR, pl.ds((i % LPR) * 16, 16)] = zvec
        return _
    lax.fori_loop(0, CH * LPR, _fill_zc, None)

    def _zero_agg(c, _):
        pltpu.sync_copy(zeroc, agg_sh.at[pl.ds(row0 + c * CH, CH)])
        return _
    lax.fori_loop(0, NCC, _zero_agg, None)
    plsc.subcore_barrier()

    rbufs = [rowbuf, rowbuf1, rowbuf2, rowbuf3,
             rowbuf4, rowbuf5, rowbuf6, rowbuf7]
    gsems = [sem, sem1, sem2, sem3, sem4, sem5, sem6, sem7]
    ssems = [ssem, ssem1, ssem2, ssem3, ssem4, ssem5, ssem6, ssem7]

    def _scatter_phase(src_ref):
        for b in range(NB):
            pltpu.async_copy(src_ref.at[cid].at[src_v.at[b]],
                             rbufs[b], gsems[b])

        def _grp(g, _):
            for b in range(NB):
                j = g * NB + b
                pltpu.make_async_copy(
                    src_ref.at[cid].at[src_v.at[j]],
                    rbufs[b], gsems[b]).wait()
                pltpu.async_copy(rbufs[b], agg_sh.at[dst_v.at[j]],
                                 ssems[b], add=True)
            for b in range(NB):
                j = g * NB + b
                pltpu.make_async_copy(
                    rbufs[b], agg_sh.at[dst_v.at[j]], ssems[b]).wait()

                @pl.when(g < NCH // NB - 1)
                def _():
                    pltpu.async_copy(
                        src_ref.at[cid].at[src_v.at[j + NB]],
                        rbufs[b], gsems[b])
            return _
        lax.fori_loop(0, NCH // NB, _grp, None)
        plsc.subcore_barrier()

    def _update_phase(last):
        def _ac(c):
            return rbufs[2 * (c % 2)]

        def _oc(c):
            return rbufs[2 * (c % 2) + 1]

        def _r0(c):
            return row0 + c * CH

        pltpu.async_copy(agg_sh.at[pl.ds(_r0(0), CH)], _ac(0), gsems[0])
        for c in range(NCC):
            ac, oc = _ac(c), _oc(c)
            pltpu.make_async_copy(
                agg_sh.at[pl.ds(_r0(c), CH)], ac, gsems[c % 2]).wait()
            if c + 1 < NCC:
                pltpu.async_copy(agg_sh.at[pl.ds(_r0(c + 1), CH)],
                                 _ac(c + 1), gsems[(c + 1) % 2])
            pltpu.async_copy(zeroc, agg_sh.at[pl.ds(_r0(c), CH)], ssems[0])
            if c >= 2:
                pltpu.make_async_copy(
                    oc, xs_hbm.at[cid, pl.ds(_r0(c - 2), CH)],
                    ssems[1 + (c % 2)]).wait()

            def _rows(r, _):
                i = c * CH + r
                ns = nstri[i, pl.ds(0, 16)]
                nv = nstri[i, pl.ds(16, 16)]
                f = jnp.where(last, 1.0, ns)
                for v in range(LPR):
                    sl = pl.ds(v * 16, 16)
                    oc[r, sl] = f * (ahc[i, sl] + nv * ac[r, sl])
                return _
            lax.fori_loop(0, CH, _rows, None)

            @pl.when(jnp.logical_not(last))
            def _():
                pltpu.async_copy(oc, xs_hbm.at[cid, pl.ds(_r0(c), CH)],
                                 ssems[1 + (c % 2)])

            @pl.when(last)
            def _():
                pltpu.async_copy(oc, out_hbm.at[cid, pl.ds(_r0(c), CH)],
                                 ssems[1 + (c % 2)])

        for c in range(NCC - 2, NCC):
            pltpu.make_async_copy(
                _oc(c), xs_hbm.at[cid, pl.ds(_r0(c), CH)],
                ssems[1 + (c % 2)]).wait()
        for c in range(NCC):
            pltpu.make_async_copy(
                zeroc, agg_sh.at[pl.ds(_r0(c), CH)], ssems[0]).wait()
        plsc.subcore_barrier()

    _scatter_phase(xs0_hbm)
    _update_phase(jnp.bool_(False))

    def _iter(k, _):
        _scatter_phase(xs_hbm)
        _update_phase(k == K - 2)
        return _
    lax.fori_loop(0, K - 1, _iter, None)


def _sc_propagate(ahh, normc, srcp, dstp, xs0):
    mesh = plsc.VectorSubcoreMesh(core_axis_name="c", subcore_axis_name="s")
    fn = functools.partial(
        pl.kernel,
        mesh=mesh,
        compiler_params=pltpu.CompilerParams(use_tc_tiling_on_sc=False),
        out_type=[
            jax.ShapeDtypeStruct((2, NN, DC), jnp.float32),
            jax.ShapeDtypeStruct((2, NN, DC), jnp.float32),
        ],
        scratch_types=[
            pltpu.VMEM((NCH, CH), jnp.int32),
            pltpu.VMEM((NCH, CH), jnp.int32),
            pltpu.VMEM((CH, DC), jnp.float32),
            pltpu.VMEM((CH, DC), jnp.float32),
            pltpu.VMEM((CH, DC), jnp.float32),
            pltpu.VMEM((CH, DC), jnp.float32),
            pltpu.VMEM((CH, DC), jnp.float32),
            pltpu.VMEM((CH, DC), jnp.float32),
            pltpu.VMEM((CH, DC), jnp.float32),
            pltpu.VMEM((CH, DC), jnp.float32),
            pltpu.VMEM((NPT, DC), jnp.float32),
            pltpu.VMEM((CH, DC), jnp.float32),
            pltpu.VMEM((NPT, 32), jnp.float32),
            pltpu.VMEM_SHARED((NN, DC), jnp.float32),
            pltpu.SemaphoreType.DMA,
            pltpu.SemaphoreType.DMA,
            pltpu.SemaphoreType.DMA,
            pltpu.SemaphoreType.DMA,
            pltpu.SemaphoreType.DMA,
            pltpu.SemaphoreType.DMA,
            pltpu.SemaphoreType.DMA,
            pltpu.SemaphoreType.DMA,
            pltpu.SemaphoreType.DMA,
            pltpu.SemaphoreType.DMA,
            pltpu.SemaphoreType.DMA,
            pltpu.SemaphoreType.DMA,
            pltpu.SemaphoreType.DMA,
            pltpu.SemaphoreType.DMA,
            pltpu.SemaphoreType.DMA,
            pltpu.SemaphoreType.DMA,
        ],
    )(_sc_propagate_body)
    return fn(ahh, normc, srcp, dstp, xs0)


def kernel(features, edge_index, W):
    src = edge_index[0].astype(jnp.int32).reshape(NS, EPT)
    dst = edge_index[1].astype(jnp.int32).reshape(NS, EPT)
    pad = ((0, 0), (0, EPAD - EPT))
    srcp = jnp.pad(src, pad, constant_values=DUMMY).reshape(NS, NCH, CH)
    dstp = jnp.pad(dst, pad, constant_values=DUMMY).reshape(NS, NCH, CH)
    srcdst = jnp.stack([srcp, dstp])

    feat_pad = jnp.pad(features, ((0, NN - N_NODES), (0, 0)))
    hp, ah = _project(feat_pad, W.T)

    cnt = _sc_degrees(srcdst)
    normc, ahh, xs0 = _prep(cnt, ah, hp)

    out_pad, _ = _sc_propagate(ahh, normc, srcp, dstp, xs0)
    return jnp.concatenate([out_pad[0], out_pad[1]], axis=1)[:N_NODES]
